# baseline (speedup 1.0000x reference)
import jax
import jax.numpy as jnp
from jax.experimental import pallas as pl
from jax.experimental.pallas import tpu as pltpu

_N = 4096
_IN_F = 128
_HEADS = 2
_HID = 32
_BI = 256


def _gat_body(vert_ref, edge_ref, w_ref, al_ref, ar_ref, out_ref,
              ghe_ref, f1_ref, f2_ref, smax_ref, gsum_ref):
    i = pl.program_id(0)

    @pl.when(i == 0)
    def _init():
        g = jnp.dot(vert_ref[...], w_ref[...], preferred_element_type=jnp.float32)
        for h in range(_HEADS):
            gh = g[:, h * _HID:(h + 1) * _HID]
            ghe_ref[h, :, 0:_HID] = gh.astype(jnp.bfloat16)
            ghe_ref[h, :, _HID:_HID + 1] = jnp.ones((_N, 1), jnp.bfloat16)
            sr = jax.lax.dot_general(
                ar_ref[...], gh, (((1,), (1,)), ((), ())),
                preferred_element_type=jnp.float32)
            s_max = jnp.max(sr)
            smax_ref[h] = s_max
            f1_ref[h:h + 1, :] = jnp.exp(sr - s_max).astype(jnp.bfloat16)
            f2_ref[h:h + 1, :] = jnp.exp(0.2 * (sr - s_max)).astype(jnp.bfloat16)
            gsum_ref[h:h + 1, :] = jnp.sum(gh, axis=0, keepdims=True) * (1.0 / _N)

    maskf = edge_ref[...]
    for h in range(_HEADS):
        g_blk = ghe_ref[h, pl.ds(i * _BI, _BI), 0:_HID].astype(jnp.float32)
        sl = jax.lax.dot_general(
            g_blk, al_ref[...], (((1,), (1,)), ((), ())),
            preferred_element_type=jnp.float32)
        s_max = smax_ref[h]
        m = jnp.maximum(sl, -s_max)
        e1 = jnp.exp(sl - m).astype(jnp.bfloat16)
        e2 = jnp.exp(0.2 * sl - m - 0.8 * s_max).astype(jnp.bfloat16)
        p = jnp.maximum(e1 * f1_ref[h:h + 1, :], e2 * f2_ref[h:h + 1, :]) * maskf
        o_ext = jnp.dot(p, ghe_ref[h], preferred_element_type=jnp.float32)
        o = o_ext[:, 0:_HID]
        d = o_ext[:, _HID:_HID + 1]
        o = jnp.where(d > 0, o / d, gsum_ref[h:h + 1, :])
        out_ref[:, h * _HID:(h + 1) * _HID] = jnp.where(o > 0, o, jnp.exp(o) - 1.0)


def kernel(vert, edge, W, a_l, a_r):
    edge_i8 = edge.astype(jnp.bfloat16)
    al2 = a_l.reshape(1, _HID)
    ar2 = a_r.reshape(1, _HID)
    return pl.pallas_call(
        _gat_body,
        grid=(_N // _BI,),
        in_specs=[
            pl.BlockSpec((_N, _IN_F), lambda i: (0, 0)),
            pl.BlockSpec((_BI, _N), lambda i: (i, 0)),
            pl.BlockSpec((_IN_F, _HEADS * _HID), lambda i: (0, 0)),
            pl.BlockSpec((1, _HID), lambda i: (0, 0)),
            pl.BlockSpec((1, _HID), lambda i: (0, 0)),
        ],
        out_specs=pl.BlockSpec((_BI, _HEADS * _HID), lambda i: (i, 0)),
        out_shape=jax.ShapeDtypeStruct((_N, _HEADS * _HID), jnp.float32),
        scratch_shapes=[
            pltpu.VMEM((_HEADS, _N, _HID + 1), jnp.bfloat16),
            pltpu.VMEM((_HEADS, _N), jnp.bfloat16),
            pltpu.VMEM((_HEADS, _N), jnp.bfloat16),
            pltpu.SMEM((_HEADS,), jnp.float32),
            pltpu.VMEM((_HEADS, _HID), jnp.float32),
        ],
    )(vert, edge_i8, W, al2, ar2)


# speedup vs baseline: 1.2332x; 1.2332x over previous
import jax
import jax.numpy as jnp
from jax.experimental import pallas as pl
from jax.experimental.pallas import tpu as pltpu

_N = 4096
_IN_F = 128
_HEADS = 2
_HID = 32
_BI = 256


def _gat_body(vert_ref, edge_ref, w_ref, al_ref, ar_ref, out_ref,
              ghe_ref, f1_ref, f2_ref, smax_ref, gsum_ref):
    i = pl.program_id(0)

    @pl.when(i == 0)
    def _init():
        g = jnp.dot(vert_ref[...], w_ref[...], preferred_element_type=jnp.float32)
        for h in range(_HEADS):
            gh = g[:, h * _HID:(h + 1) * _HID]
            ghe_ref[h, :, 0:_HID] = gh.astype(jnp.bfloat16)
            ghe_ref[h, :, _HID:_HID + 1] = jnp.ones((_N, 1), jnp.bfloat16)
            sr = jax.lax.dot_general(
                ar_ref[...], gh, (((1,), (1,)), ((), ())),
                preferred_element_type=jnp.float32)
            s_max = jnp.max(sr)
            smax_ref[h] = s_max
            f1_ref[h:h + 1, :] = jnp.exp(sr - s_max).astype(jnp.bfloat16)
            f2_ref[h:h + 1, :] = jnp.exp(0.2 * (sr - s_max)).astype(jnp.bfloat16)
            gsum_ref[h:h + 1, :] = jnp.sum(gh, axis=0, keepdims=True) * (1.0 / _N)

    maskf = edge_ref[...].astype(jnp.bfloat16)
    for h in range(_HEADS):
        g_blk = ghe_ref[h, pl.ds(i * _BI, _BI), 0:_HID].astype(jnp.float32)
        sl = jax.lax.dot_general(
            g_blk, al_ref[...], (((1,), (1,)), ((), ())),
            preferred_element_type=jnp.float32)
        s_max = smax_ref[h]
        m = jnp.maximum(sl, -s_max)
        e1 = jnp.exp(sl - m).astype(jnp.bfloat16)
        e2 = jnp.exp(0.2 * sl - m - 0.8 * s_max).astype(jnp.bfloat16)
        p = jnp.maximum(e1 * f1_ref[h:h + 1, :], e2 * f2_ref[h:h + 1, :]) * maskf
        o_ext = jnp.dot(p, ghe_ref[h], preferred_element_type=jnp.float32)
        o = o_ext[:, 0:_HID]
        d = o_ext[:, _HID:_HID + 1]
        o = jnp.where(d > 0, o / d, gsum_ref[h:h + 1, :])
        o = jnp.where(o > 0, o, jnp.exp(o) - 1.0)
        out_ref[h * _HID:(h + 1) * _HID, :] = o.T


def kernel(vert, edge, W, a_l, a_r):
    edge_i8 = edge.astype(jnp.int8)
    al2 = a_l.reshape(1, _HID)
    ar2 = a_r.reshape(1, _HID)
    return pl.pallas_call(
        _gat_body,
        grid=(_N // _BI,),
        in_specs=[
            pl.BlockSpec((_N, _IN_F), lambda i: (0, 0)),
            pl.BlockSpec((_BI, _N), lambda i: (i, 0)),
            pl.BlockSpec((_IN_F, _HEADS * _HID), lambda i: (0, 0)),
            pl.BlockSpec((1, _HID), lambda i: (0, 0)),
            pl.BlockSpec((1, _HID), lambda i: (0, 0)),
        ],
        out_specs=pl.BlockSpec((_HEADS * _HID, _BI), lambda i: (0, i)),
        out_shape=jax.ShapeDtypeStruct((_HEADS * _HID, _N), jnp.float32),
        scratch_shapes=[
            pltpu.VMEM((_HEADS, _N, _HID + 1), jnp.bfloat16),
            pltpu.VMEM((_HEADS, _N), jnp.bfloat16),
            pltpu.VMEM((_HEADS, _N), jnp.bfloat16),
            pltpu.SMEM((_HEADS,), jnp.float32),
            pltpu.VMEM((_HEADS, _HID), jnp.float32),
        ],
    )(vert, edge_i8, W, al2, ar2).T
